# baseline (speedup 1.0000x reference)
import functools

import jax
import jax.numpy as jnp
from jax import lax
from jax.experimental import pallas as pl
from jax.experimental.pallas import tpu as pltpu
from jax.experimental.pallas import tpu_sc as plsc

D = 128
SCALE = float(D) ** 0.5
NC = 2
NS = 16
NW = NC * NS
CHUNK = 128
LANES = 16


NBUF = 2


@functools.lru_cache(maxsize=None)
def _emb_kernel(B):
    n_chunks = B // (NW * CHUNK)
    n_rounds = n_chunks // NBUF
    assert n_chunks % NBUF == 0 and n_rounds >= 3
    mesh = plsc.VectorSubcoreMesh(core_axis_name="c", subcore_axis_name="s")

    @functools.partial(
        pl.kernel,
        mesh=mesh,
        out_type=jax.ShapeDtypeStruct((B, D), jnp.float32),
        scratch_types=[
            pltpu.VMEM((n_chunks, CHUNK), jnp.int32),
        ]
        + [pltpu.VMEM((CHUNK, D), jnp.float32)] * (2 * NBUF)
        + [pltpu.SemaphoreType.DMA] * (2 * NBUF),
    )
    def k(idx_hbm, table_hbm, out_hbm, idx_v, g0, g1, s0, s1,
          gsem0, gsem1, ssem0, ssem1):
        gbuf, sbuf = (g0, g1), (s0, s1)
        gsem, ssem = (gsem0, gsem1), (ssem0, ssem1)
        wid = lax.axis_index("s") * NC + lax.axis_index("c")
        base = wid * (n_chunks * CHUNK)
        pltpu.sync_copy(idx_hbm.at[wid], idx_v)

        def g_start(j, b):
            pltpu.async_copy(table_hbm.at[idx_v.at[j]], gbuf[b], gsem[b])

        def g_wait(b):
            pltpu.make_async_copy(
                table_hbm.at[idx_v.at[0]], gbuf[b], gsem[b]).wait()

        def s_start(j, b):
            pltpu.async_copy(
                sbuf[b], out_hbm.at[pl.ds(base + j * CHUNK, CHUNK)], ssem[b])

        def s_wait(b):
            pltpu.make_async_copy(
                sbuf[b], out_hbm.at[pl.ds(base, CHUNK)], ssem[b]).wait()

        def scale(b):
            @plsc.parallel_loop(0, CHUNK, 1, unroll=2)
            def _(r):
                for c in range(D // LANES):
                    sl = pl.ds(c * LANES, LANES)
                    sbuf[b][r, sl] = gbuf[b][r, sl] * SCALE

        for b in range(NBUF):
            g_start(b, b)
        for b in range(NBUF):
            g_wait(b)
            scale(b)
            g_start(NBUF + b, b)
            s_start(b, b)

        def outer(t, carry):
            for b in range(NBUF):
                j = t * NBUF + b
                g_wait(b)
                s_wait(b)
                scale(b)
                g_start(j + NBUF, b)
                s_start(j, b)
            return carry

        lax.fori_loop(1, n_rounds - 1, outer, 0)

        for b in range(NBUF):
            g_wait(b)
            s_wait(b)
            scale(b)
            s_start(n_chunks - NBUF + b, b)
        for b in range(NBUF):
            s_wait(b)

    return k


def kernel(inputs, table):
    bt, s = inputs.shape
    b = bt * s
    idx = inputs.reshape(NW, b // (NW * CHUNK), CHUNK).astype(jnp.int32)
    out = _emb_kernel(b)(idx, table)
    return out.reshape(bt, s, D)


# speedup vs baseline: 1.0175x; 1.0175x over previous
import functools

import jax
import jax.numpy as jnp
from jax import lax
from jax.experimental import pallas as pl
from jax.experimental.pallas import tpu as pltpu
from jax.experimental.pallas import tpu_sc as plsc

D = 128
SCALE = float(D) ** 0.5
NC = 2
NS = 16
NW = NC * NS
CHUNK = 64
LANES = 16


NBUF = 4


@functools.lru_cache(maxsize=None)
def _emb_kernel(B):
    n_chunks = B // (NW * CHUNK)
    n_rounds = n_chunks // NBUF
    assert n_chunks % NBUF == 0 and n_rounds >= 3
    mesh = plsc.VectorSubcoreMesh(core_axis_name="c", subcore_axis_name="s")

    @functools.partial(
        pl.kernel,
        mesh=mesh,
        out_type=jax.ShapeDtypeStruct((B, D), jnp.float32),
        scratch_types=[
            pltpu.VMEM((n_chunks, CHUNK), jnp.int32),
        ]
        + [pltpu.VMEM((CHUNK, D), jnp.float32)] * (2 * NBUF)
        + [pltpu.SemaphoreType.DMA] * (2 * NBUF),
    )
    def k(idx_hbm, table_hbm, out_hbm, idx_v, *bufs):
        gbuf = bufs[0:NBUF]
        sbuf = bufs[NBUF:2 * NBUF]
        gsem = bufs[2 * NBUF:3 * NBUF]
        ssem = bufs[3 * NBUF:4 * NBUF]
        wid = lax.axis_index("s") * NC + lax.axis_index("c")
        base = wid * (n_chunks * CHUNK)
        pltpu.sync_copy(idx_hbm.at[wid], idx_v)

        def g_start(j, b):
            pltpu.async_copy(table_hbm.at[idx_v.at[j]], gbuf[b], gsem[b])

        def g_wait(b):
            pltpu.make_async_copy(
                table_hbm.at[idx_v.at[0]], gbuf[b], gsem[b]).wait()

        def s_start(j, b):
            pltpu.async_copy(
                sbuf[b], out_hbm.at[pl.ds(base + j * CHUNK, CHUNK)], ssem[b])

        def s_wait(b):
            pltpu.make_async_copy(
                sbuf[b], out_hbm.at[pl.ds(base, CHUNK)], ssem[b]).wait()

        def scale(b):
            def row_body(r, c2):
                for c in range(D // LANES):
                    sl = pl.ds(c * LANES, LANES)
                    sbuf[b][r, sl] = gbuf[b][r, sl] * SCALE
                return c2

            lax.fori_loop(0, CHUNK, row_body, 0)

        for b in range(NBUF):
            g_start(b, b)
        for b in range(NBUF):
            g_wait(b)
            scale(b)
            g_start(NBUF + b, b)
            s_start(b, b)

        def outer(t, carry):
            for b in range(NBUF):
                j = t * NBUF + b
                g_wait(b)
                s_wait(b)
                scale(b)
                g_start(j + NBUF, b)
                s_start(j, b)
            return carry

        lax.fori_loop(1, n_rounds - 1, outer, 0)

        for b in range(NBUF):
            g_wait(b)
            s_wait(b)
            scale(b)
            s_start(n_chunks - NBUF + b, b)
        for b in range(NBUF):
            s_wait(b)

    return k


def kernel(inputs, table):
    bt, s = inputs.shape
    b = bt * s
    idx = inputs.reshape(NW, b // (NW * CHUNK), CHUNK).astype(jnp.int32)
    out = _emb_kernel(b)(idx, table)
    return out.reshape(bt, s, D)
